# Initial kernel scaffold; baseline (speedup 1.0000x reference)
import jax
import jax.numpy as jnp
from jax.experimental import pallas as pl


def kernel(x, edge_index, batch, edge_attr, W1, b1, W2, b2, eps, We, be, gamma, beta, Wm1, bm1, Wm2, bm2, Wm3, bm3):
    raise NotImplementedError("write your pallas kernel here")


# speedup vs baseline: 2.4394x; 2.4394x over previous
import jax
import jax.numpy as jnp
from jax import lax
from jax.experimental import pallas as pl
from jax.experimental.pallas import tpu as pltpu
from jax.experimental.pallas import tpu_sc as plsc

N = 10000
E = 320000
D = 128
ED = 4
NLAYER = 5
G = 64

NC = 2
NS = 16
HD = D // NC
EPT = E // NS
BB = 80
NBLK = EPT // BB
RPS = 624
DC = HD // 16


def _sc_edge_body(h_hbm, src_hbm, dst_hbm, ea_hbm, wb_hbm, out_hbm,
                  src_v, dst_v, ea_v, gbuf, mbuf, wb_v, aggr_sh):
    cid = lax.axis_index("c")
    sid = lax.axis_index("s")

    pltpu.sync_copy(src_hbm.at[sid], src_v)
    pltpu.sync_copy(dst_hbm.at[sid], dst_v)
    pltpu.sync_copy(wb_hbm.at[cid], wb_v)

    @plsc.parallel_loop(0, NBLK * BB // 16, 1, unroll=8)
    def _ix(i):
        r = i // (BB // 16)
        c16 = (i % (BB // 16)) * 16
        src_v[r, pl.ds(c16, 16)] = src_v[r, pl.ds(c16, 16)] * 2 + cid

    @plsc.parallel_loop(0, BB, 1, unroll=4)
    def _zero(i):
        for c in range(DC):
            mbuf[i, pl.ds(c * 16, 16)] = jnp.zeros((16,), jnp.float32)

    r0 = pl.multiple_of(sid * RPS, 8)
    for t in range(7):
        pltpu.sync_copy(mbuf, aggr_sh.at[pl.ds(r0 + t * BB, BB)])
    pltpu.sync_copy(mbuf.at[pl.ds(0, RPS - 7 * BB)],
                    aggr_sh.at[pl.ds(r0 + 7 * BB, RPS - 7 * BB)])

    @pl.when(sid == NS - 1)
    def _zero_tail():
        pltpu.sync_copy(mbuf.at[pl.ds(0, 16)],
                        aggr_sh.at[pl.ds(NS * RPS, N - NS * RPS)])

    plsc.subcore_barrier()

    we_rows = [[wb_v[0, pl.ds(r * HD + c * 16, 16)] for c in range(DC)]
               for r in range(ED)]
    be_row = [wb_v[0, pl.ds(ED * HD + c * 16, 16)] for c in range(DC)]

    def blk_body(blk, carry):
        ea_off = pl.multiple_of((sid * EPT + blk * BB) * ED, 8)
        pltpu.sync_copy(ea_hbm.at[pl.ds(ea_off, BB * ED + 64)], ea_v)
        pltpu.sync_copy(h_hbm.at[src_v.at[blk]], gbuf)

        @plsc.parallel_loop(0, BB, 1, unroll=2)
        def _edge(jj):
            av = ea_v[pl.ds(jj * ED, 16)]
            a0 = av[0]
            a1 = av[1]
            a2 = av[2]
            a3 = av[3]
            for c in range(DC):
                m = gbuf[jj, pl.ds(c * 16, 16)] + be_row[c]
                m = m + a0 * we_rows[0][c]
                m = m + a1 * we_rows[1][c]
                m = m + a2 * we_rows[2][c]
                m = m + a3 * we_rows[3][c]
                mbuf[jj, pl.ds(c * 16, 16)] = jnp.maximum(m, 0.0)

        pltpu.sync_copy(mbuf, aggr_sh.at[dst_v.at[blk]], add=True)
        return carry

    lax.fori_loop(0, NBLK, blk_body, 0)
    plsc.subcore_barrier()

    for t in range(7):
        pltpu.sync_copy(aggr_sh.at[pl.ds(r0 + t * BB, BB)],
                        out_hbm.at[cid, pl.ds(r0 + t * BB, BB)])
    pltpu.sync_copy(aggr_sh.at[pl.ds(r0 + 7 * BB, RPS - 7 * BB)],
                    out_hbm.at[cid, pl.ds(r0 + 7 * BB, RPS - 7 * BB)])

    @pl.when(sid == NS - 1)
    def _wb_tail():
        pltpu.sync_copy(aggr_sh.at[pl.ds(NS * RPS, N - NS * RPS)],
                        out_hbm.at[cid, pl.ds(NS * RPS, N - NS * RPS)])


@jax.jit
def _sc_edge(h2d, src3d, dst3d, ea_flat, wb):
    mesh = plsc.VectorSubcoreMesh(core_axis_name="c", subcore_axis_name="s",
                                  num_cores=NC, num_subcores=NS)
    return pl.kernel(
        _sc_edge_body,
        out_type=jax.ShapeDtypeStruct((NC, N, HD), jnp.float32),
        mesh=mesh,
        compiler_params=pltpu.CompilerParams(use_tc_tiling_on_sc=False),
        scratch_types=[
            pltpu.VMEM((NBLK, BB), jnp.int32),
            pltpu.VMEM((NBLK, BB), jnp.int32),
            pltpu.VMEM((BB * ED + 64,), jnp.float32),
            pltpu.VMEM((BB, HD), jnp.float32),
            pltpu.VMEM((BB, HD), jnp.float32),
            pltpu.VMEM((1, (ED + 1) * HD), jnp.float32),
            pltpu.VMEM_SHARED((N, HD), jnp.float32),
        ],
    )(h2d, src3d, dst3d, ea_flat, wb)


def _tc_layer_body(eps_ref, h_ref, a_ref, w1_ref, b1_ref, w2_ref,
                   b2_ref, g_ref, bt_ref, out_ref):
    h = h_ref[...]
    s = 1.0 + eps_ref[0]
    aggr = jnp.concatenate([a_ref[0], a_ref[1]], axis=-1)
    h2 = s * h + aggr
    t = jnp.dot(h2, w1_ref[...], preferred_element_type=jnp.float32)
    t = jnp.maximum(t + b1_ref[...], 0.0)
    t2 = jnp.dot(t, w2_ref[...], preferred_element_type=jnp.float32)
    t2 = t2 + b2_ref[...]
    mean = jnp.mean(t2, axis=0, keepdims=True)
    var = jnp.mean((t2 - mean) * (t2 - mean), axis=0, keepdims=True)
    bn = g_ref[...] * (t2 - mean) * lax.rsqrt(var + 1e-5) + bt_ref[...]
    out_ref[...] = jnp.maximum(bn, 0.0) + h


@jax.jit
def _tc_layer(eps_i, h, parts, w1, b1, w2, b2, g, bt):
    return pl.pallas_call(
        _tc_layer_body,
        out_shape=jax.ShapeDtypeStruct((N, D), jnp.float32),
        in_specs=[pl.BlockSpec(memory_space=pltpu.SMEM),
                  pl.BlockSpec((N, D), lambda: (0, 0)),
                  pl.BlockSpec((NC, N, HD), lambda: (0, 0, 0)),
                  pl.BlockSpec((D, D), lambda: (0, 0)),
                  pl.BlockSpec((1, D), lambda: (0, 0)),
                  pl.BlockSpec((D, D), lambda: (0, 0)),
                  pl.BlockSpec((1, D), lambda: (0, 0)),
                  pl.BlockSpec((1, D), lambda: (0, 0)),
                  pl.BlockSpec((1, D), lambda: (0, 0))],
        out_specs=pl.BlockSpec((N, D), lambda: (0, 0)),
    )(eps_i, h, parts, w1, b1, w2, b2, g, bt)


def _tc_final_body(batch_ref, h_ref, w1_ref, b1_ref, w2_ref, b2_ref, w3_ref,
                   b3_ref, out_ref):
    b = batch_ref[...]
    gids = lax.broadcasted_iota(jnp.int32, (G, 1), 0)
    onehot = (b == gids).astype(jnp.float32)
    sums = jnp.dot(onehot, h_ref[...], preferred_element_type=jnp.float32)
    cnt = jnp.sum(onehot, axis=1, keepdims=True)
    pooled = sums / jnp.maximum(cnt, 1.0)
    o = jnp.dot(pooled, w1_ref[...], preferred_element_type=jnp.float32)
    o = jnp.maximum(o + b1_ref[...], 0.0)
    o = jnp.dot(o, w2_ref[...], preferred_element_type=jnp.float32)
    o = jnp.maximum(o + b2_ref[...], 0.0)
    o = jnp.dot(o, w3_ref[...], preferred_element_type=jnp.float32)
    out_ref[...] = o + b3_ref[...]


@jax.jit
def _tc_final(batch2d, h, w1, b1, w2, b2, w3, b3):
    return pl.pallas_call(
        _tc_final_body,
        out_shape=jax.ShapeDtypeStruct((G, 1), jnp.float32),
        in_specs=[pl.BlockSpec((1, N), lambda: (0, 0)),
                  pl.BlockSpec((N, D), lambda: (0, 0)),
                  pl.BlockSpec((D, D), lambda: (0, 0)),
                  pl.BlockSpec((1, D), lambda: (0, 0)),
                  pl.BlockSpec((D, D), lambda: (0, 0)),
                  pl.BlockSpec((1, D), lambda: (0, 0)),
                  pl.BlockSpec((D, 1), lambda: (0, 0)),
                  pl.BlockSpec((1, 1), lambda: (0, 0))],
        out_specs=pl.BlockSpec((G, 1), lambda: (0, 0)),
    )(batch2d, h, w1, b1, w2, b2, w3, b3)


def kernel(x, edge_index, batch, edge_attr, W1, b1, W2, b2, eps, We, be,
           gamma, beta, Wm1, bm1, Wm2, bm2, Wm3, bm3):
    src3d = edge_index[0].astype(jnp.int32).reshape(NS, NBLK, BB)
    dst3d = edge_index[1].astype(jnp.int32).reshape(NS, NBLK, BB)
    ea_flat = jnp.pad(edge_attr.reshape(E * ED), (0, 128))
    batch2d = batch.astype(jnp.int32).reshape(1, N)

    h = x
    for i in range(NLAYER):
        wb = jnp.stack([
            jnp.concatenate([We[i][:, c * HD:(c + 1) * HD].reshape(-1),
                             be[i][c * HD:(c + 1) * HD]]).reshape(1, -1)
            for c in range(NC)])
        parts = _sc_edge(h.reshape(NC * N, HD), src3d, dst3d, ea_flat, wb)
        h = _tc_layer(eps[i].reshape(1), h, parts,
                      W1[i], b1[i].reshape(1, D), W2[i], b2[i].reshape(1, D),
                      gamma[i].reshape(1, D), beta[i].reshape(1, D))
    return _tc_final(batch2d, h, Wm1, bm1.reshape(1, D), Wm2,
                     bm2.reshape(1, D), Wm3, bm3.reshape(1, 1))


# speedup vs baseline: 5.6410x; 2.3125x over previous
import jax
import jax.numpy as jnp
from jax import lax
from jax.experimental import pallas as pl
from jax.experimental.pallas import tpu as pltpu
from jax.experimental.pallas import tpu_sc as plsc

N = 10000
E = 320000
D = 128
ED = 4
NLAYER = 5
G = 64

NC = 2
NS = 16
HD = D // NC
EPT = E // NS
BB = 80
NBLK = EPT // BB
RPS = 624
DC = HD // 16


def _sc_edge_body(h_hbm, src_hbm, dst_hbm, ea_hbm, wb_hbm, out_hbm,
                  src_v, dst_v, ea_v, gbuf, mbuf, wb_v, aggr_sh,
                  gsem0, gsem1, esem0, esem1, ssem0, ssem1):
    cid = lax.axis_index("c")
    sid = lax.axis_index("s")
    gsems = (gsem0, gsem1)
    esems = (esem0, esem1)
    ssems = (ssem0, ssem1)

    pltpu.sync_copy(src_hbm.at[sid], src_v)
    pltpu.sync_copy(dst_hbm.at[sid], dst_v)
    pltpu.sync_copy(wb_hbm.at[cid], wb_v)

    @plsc.parallel_loop(0, NBLK * BB // 16, 1, unroll=8)
    def _ix(i):
        r = i // (BB // 16)
        c16 = (i % (BB // 16)) * 16
        src_v[r, pl.ds(c16, 16)] = src_v[r, pl.ds(c16, 16)] * 2 + cid

    @plsc.parallel_loop(0, BB, 1, unroll=4)
    def _zero(i):
        for c in range(DC):
            mbuf[0, i, pl.ds(c * 16, 16)] = jnp.zeros((16,), jnp.float32)

    r0 = pl.multiple_of(sid * RPS, 8)
    for t in range(7):
        pltpu.sync_copy(mbuf.at[0], aggr_sh.at[pl.ds(r0 + t * BB, BB)])
    pltpu.sync_copy(mbuf.at[0, pl.ds(0, RPS - 7 * BB)],
                    aggr_sh.at[pl.ds(r0 + 7 * BB, RPS - 7 * BB)])

    @pl.when(sid == NS - 1)
    def _zero_tail():
        pltpu.sync_copy(mbuf.at[0, pl.ds(0, 16)],
                        aggr_sh.at[pl.ds(NS * RPS, N - NS * RPS)])

    plsc.subcore_barrier()

    we_rows = [[wb_v[0, pl.ds(r * HD + c * 16, 16)] for c in range(DC)]
               for r in range(ED)]
    be_row = [wb_v[0, pl.ds(ED * HD + c * 16, 16)] for c in range(DC)]

    def issue(b, par):
        ea_off = pl.multiple_of((sid * EPT + b * BB) * ED, 8)
        pltpu.async_copy(ea_hbm.at[pl.ds(ea_off, BB * ED + 64)],
                         ea_v.at[par], esems[par])
        pltpu.async_copy(h_hbm.at[src_v.at[b]], gbuf.at[par], gsems[par])

    def wait_inputs(b, par):
        ea_off = pl.multiple_of((sid * EPT + b * BB) * ED, 8)
        pltpu.make_async_copy(ea_hbm.at[pl.ds(ea_off, BB * ED + 64)],
                              ea_v.at[par], esems[par]).wait()
        pltpu.make_async_copy(h_hbm.at[src_v.at[b]], gbuf.at[par],
                              gsems[par]).wait()

    def compute(par):
        @plsc.parallel_loop(0, BB, 1, unroll=2)
        def _edge(jj):
            av = ea_v[par, pl.ds(jj * ED, 16)]
            a0 = av[0]
            a1 = av[1]
            a2 = av[2]
            a3 = av[3]
            for c in range(DC):
                m = gbuf[par, jj, pl.ds(c * 16, 16)] + be_row[c]
                m = m + a0 * we_rows[0][c]
                m = m + a1 * we_rows[1][c]
                m = m + a2 * we_rows[2][c]
                m = m + a3 * we_rows[3][c]
                mbuf[par, jj, pl.ds(c * 16, 16)] = jnp.maximum(m, 0.0)

    def scatter(b, par):
        pltpu.async_copy(mbuf.at[par], aggr_sh.at[dst_v.at[b]],
                         ssems[par], add=True)

    def wait_scatter(b, par):
        pltpu.make_async_copy(mbuf.at[par], aggr_sh.at[dst_v.at[b]],
                              ssems[par]).wait()

    issue(0, 0)

    def pipe_body(i, carry):
        b0 = i * 2
        b1 = b0 + 1
        issue(b1, 1)
        wait_inputs(b0, 0)

        @pl.when(i > 0)
        def _ws0():
            wait_scatter(b0 - 2, 0)

        compute(0)
        scatter(b0, 0)

        @pl.when(i < NBLK // 2 - 1)
        def _iss0():
            issue(b0 + 2, 0)

        wait_inputs(b1, 1)

        @pl.when(i > 0)
        def _ws1():
            wait_scatter(b1 - 2, 1)

        compute(1)
        scatter(b1, 1)
        return carry

    lax.fori_loop(0, NBLK // 2, pipe_body, 0)
    wait_scatter(NBLK - 2, 0)
    wait_scatter(NBLK - 1, 1)
    plsc.subcore_barrier()

    for t in range(7):
        pltpu.sync_copy(aggr_sh.at[pl.ds(r0 + t * BB, BB)],
                        out_hbm.at[cid, pl.ds(r0 + t * BB, BB)])
    pltpu.sync_copy(aggr_sh.at[pl.ds(r0 + 7 * BB, RPS - 7 * BB)],
                    out_hbm.at[cid, pl.ds(r0 + 7 * BB, RPS - 7 * BB)])

    @pl.when(sid == NS - 1)
    def _wb_tail():
        pltpu.sync_copy(aggr_sh.at[pl.ds(NS * RPS, N - NS * RPS)],
                        out_hbm.at[cid, pl.ds(NS * RPS, N - NS * RPS)])


@jax.jit
def _sc_edge(h2d, src3d, dst3d, ea_flat, wb):
    mesh = plsc.VectorSubcoreMesh(core_axis_name="c", subcore_axis_name="s",
                                  num_cores=NC, num_subcores=NS)
    return pl.kernel(
        _sc_edge_body,
        out_type=jax.ShapeDtypeStruct((NC, N, HD), jnp.float32),
        mesh=mesh,
        compiler_params=pltpu.CompilerParams(use_tc_tiling_on_sc=False),
        scratch_types=[
            pltpu.VMEM((NBLK, BB), jnp.int32),
            pltpu.VMEM((NBLK, BB), jnp.int32),
            pltpu.VMEM((2, BB * ED + 64), jnp.float32),
            pltpu.VMEM((2, BB, HD), jnp.float32),
            pltpu.VMEM((2, BB, HD), jnp.float32),
            pltpu.VMEM((1, (ED + 1) * HD), jnp.float32),
            pltpu.VMEM_SHARED((N, HD), jnp.float32),
            pltpu.SemaphoreType.DMA,
            pltpu.SemaphoreType.DMA,
            pltpu.SemaphoreType.DMA,
            pltpu.SemaphoreType.DMA,
            pltpu.SemaphoreType.DMA,
            pltpu.SemaphoreType.DMA,
        ],
    )(h2d, src3d, dst3d, ea_flat, wb)


def _tc_layer_body(eps_ref, h_ref, a_ref, w1_ref, b1_ref, w2_ref,
                   b2_ref, g_ref, bt_ref, out_ref):
    h = h_ref[...]
    s = 1.0 + eps_ref[0]
    aggr = jnp.concatenate([a_ref[0], a_ref[1]], axis=-1)
    h2 = s * h + aggr
    t = jnp.dot(h2, w1_ref[...], preferred_element_type=jnp.float32)
    t = jnp.maximum(t + b1_ref[...], 0.0)
    t2 = jnp.dot(t, w2_ref[...], preferred_element_type=jnp.float32)
    t2 = t2 + b2_ref[...]
    mean = jnp.mean(t2, axis=0, keepdims=True)
    var = jnp.mean((t2 - mean) * (t2 - mean), axis=0, keepdims=True)
    bn = g_ref[...] * (t2 - mean) * lax.rsqrt(var + 1e-5) + bt_ref[...]
    out_ref[...] = jnp.maximum(bn, 0.0) + h


@jax.jit
def _tc_layer(eps_i, h, parts, w1, b1, w2, b2, g, bt):
    return pl.pallas_call(
        _tc_layer_body,
        out_shape=jax.ShapeDtypeStruct((N, D), jnp.float32),
        in_specs=[pl.BlockSpec(memory_space=pltpu.SMEM),
                  pl.BlockSpec((N, D), lambda: (0, 0)),
                  pl.BlockSpec((NC, N, HD), lambda: (0, 0, 0)),
                  pl.BlockSpec((D, D), lambda: (0, 0)),
                  pl.BlockSpec((1, D), lambda: (0, 0)),
                  pl.BlockSpec((D, D), lambda: (0, 0)),
                  pl.BlockSpec((1, D), lambda: (0, 0)),
                  pl.BlockSpec((1, D), lambda: (0, 0)),
                  pl.BlockSpec((1, D), lambda: (0, 0))],
        out_specs=pl.BlockSpec((N, D), lambda: (0, 0)),
    )(eps_i, h, parts, w1, b1, w2, b2, g, bt)


def _tc_final_body(batch_ref, h_ref, w1_ref, b1_ref, w2_ref, b2_ref, w3_ref,
                   b3_ref, out_ref):
    b = batch_ref[...]
    gids = lax.broadcasted_iota(jnp.int32, (G, 1), 0)
    onehot = (b == gids).astype(jnp.float32)
    sums = jnp.dot(onehot, h_ref[...], preferred_element_type=jnp.float32)
    cnt = jnp.sum(onehot, axis=1, keepdims=True)
    pooled = sums / jnp.maximum(cnt, 1.0)
    o = jnp.dot(pooled, w1_ref[...], preferred_element_type=jnp.float32)
    o = jnp.maximum(o + b1_ref[...], 0.0)
    o = jnp.dot(o, w2_ref[...], preferred_element_type=jnp.float32)
    o = jnp.maximum(o + b2_ref[...], 0.0)
    o = jnp.dot(o, w3_ref[...], preferred_element_type=jnp.float32)
    out_ref[...] = o + b3_ref[...]


@jax.jit
def _tc_final(batch2d, h, w1, b1, w2, b2, w3, b3):
    return pl.pallas_call(
        _tc_final_body,
        out_shape=jax.ShapeDtypeStruct((G, 1), jnp.float32),
        in_specs=[pl.BlockSpec((1, N), lambda: (0, 0)),
                  pl.BlockSpec((N, D), lambda: (0, 0)),
                  pl.BlockSpec((D, D), lambda: (0, 0)),
                  pl.BlockSpec((1, D), lambda: (0, 0)),
                  pl.BlockSpec((D, D), lambda: (0, 0)),
                  pl.BlockSpec((1, D), lambda: (0, 0)),
                  pl.BlockSpec((D, 1), lambda: (0, 0)),
                  pl.BlockSpec((1, 1), lambda: (0, 0))],
        out_specs=pl.BlockSpec((G, 1), lambda: (0, 0)),
    )(batch2d, h, w1, b1, w2, b2, w3, b3)


def kernel(x, edge_index, batch, edge_attr, W1, b1, W2, b2, eps, We, be,
           gamma, beta, Wm1, bm1, Wm2, bm2, Wm3, bm3):
    src3d = edge_index[0].astype(jnp.int32).reshape(NS, NBLK, BB)
    dst3d = edge_index[1].astype(jnp.int32).reshape(NS, NBLK, BB)
    ea_flat = jnp.pad(edge_attr.reshape(E * ED), (0, 128))
    batch2d = batch.astype(jnp.int32).reshape(1, N)

    h = x
    for i in range(NLAYER):
        wb = jnp.stack([
            jnp.concatenate([We[i][:, c * HD:(c + 1) * HD].reshape(-1),
                             be[i][c * HD:(c + 1) * HD]]).reshape(1, -1)
            for c in range(NC)])
        parts = _sc_edge(h.reshape(NC * N, HD), src3d, dst3d, ea_flat, wb)
        h = _tc_layer(eps[i].reshape(1), h, parts,
                      W1[i], b1[i].reshape(1, D), W2[i], b2[i].reshape(1, D),
                      gamma[i].reshape(1, D), beta[i].reshape(1, D))
    return _tc_final(batch2d, h, Wm1, bm1.reshape(1, D), Wm2,
                     bm2.reshape(1, D), Wm3, bm3.reshape(1, 1))
